# baseline (speedup 1.0000x reference)
import jax
import jax.numpy as jnp
from jax.experimental import pallas as pl
from jax.experimental.pallas import tpu as pltpu

_BK = 128


def _gcnn_kernel(adj_hbm, x_ref, w_ref, b_ref, out_ref, abuf, ab_ref, *sems):
    n = adj_hbm.shape[0]
    num_blocks = n // _BK

    def block_copy(k):
        rows = pl.ds(k * _BK, _BK)
        return pltpu.make_async_copy(adj_hbm.at[rows, :], abuf.at[rows, :],
                                     sems[k])

    for k in range(num_blocks):
        block_copy(k).start()

    cs = jnp.zeros((1, n), jnp.int32)
    for k in range(num_blocks):
        block_copy(k).wait()
        blk = abuf[k * _BK:(k + 1) * _BK, :]
        cs = cs + jnp.sum(blk, axis=0, keepdims=True)
        ab_ref[k * _BK:(k + 1) * _BK, :] = blk.astype(jnp.bfloat16)

    d = jax.lax.rsqrt(cs.astype(jnp.float32) + 1.0)
    dc = d.reshape(-1, 1)
    xw = jnp.dot(x_ref[...], w_ref[...], preferred_element_type=jnp.float32)
    y = xw * dc
    zt = jnp.dot(y.astype(jnp.bfloat16).T, ab_ref[...],
                 preferred_element_type=jnp.float32)
    out_ref[...] = (zt.T + y) * dc + b_ref[...]


def kernel(batch_inputs, batch_graph, W, b):
    n, f = batch_inputs.shape
    fo = W.shape[1]
    num_blocks = n // _BK
    return pl.pallas_call(
        _gcnn_kernel,
        in_specs=[
            pl.BlockSpec(memory_space=pl.ANY),
            pl.BlockSpec((n, f), lambda: (0, 0)),
            pl.BlockSpec((f, fo), lambda: (0, 0)),
            pl.BlockSpec((1, fo), lambda: (0, 0)),
        ],
        out_specs=pl.BlockSpec((n, fo), lambda: (0, 0)),
        scratch_shapes=[
            pltpu.VMEM((n, n), jnp.int32),
            pltpu.VMEM((n, n), jnp.bfloat16),
        ] + [pltpu.SemaphoreType.DMA] * num_blocks,
        out_shape=jax.ShapeDtypeStruct((n, fo), batch_inputs.dtype),
    )(batch_graph, batch_inputs, W, b.reshape(1, -1))


# speedup vs baseline: 1.1996x; 1.1996x over previous
import jax
import jax.numpy as jnp
from jax.experimental import pallas as pl


def _gcnn_kernel(adj_ref, x_ref, w_ref, b_ref, out_ref):
    ai = adj_ref[...]
    deg = jnp.sum(ai, axis=0, keepdims=True)
    d = jax.lax.rsqrt(deg.astype(jnp.float32) + 1.0)
    dc = d.reshape(-1, 1)
    xw = jnp.dot(x_ref[...], w_ref[...], preferred_element_type=jnp.float32)
    y = xw * dc
    zt = jnp.dot(y.astype(jnp.bfloat16).T, ai.astype(jnp.bfloat16),
                 preferred_element_type=jnp.float32)
    out_ref[...] = (zt.T + y) * dc + b_ref[...]


def kernel(batch_inputs, batch_graph, W, b):
    n, f = batch_inputs.shape
    return pl.pallas_call(
        _gcnn_kernel,
        out_shape=jax.ShapeDtypeStruct((n, W.shape[1]), batch_inputs.dtype),
    )(batch_graph, batch_inputs, W, b.reshape(1, -1))
